# baseline (speedup 1.0000x reference)
import functools

import jax
import jax.numpy as jnp
from jax import lax
from jax.experimental import pallas as pl
from jax.experimental.pallas import tpu as pltpu
from jax.experimental.pallas import tpu_sc as plsc

N = 10000
D = 128
ODIM = 64
NP = 10240
NC, NS = 2, 16
NW = NC * NS
CHUNK = 128
CPW = 80
SUB = 64
NSUB = 160
EPW = CPW * CHUNK
EP = NW * EPW
RPT = NP // NS
RB = 2048
GRID = NP // RB

_sc_mesh = plsc.VectorSubcoreMesh(
    core_axis_name="c", subcore_axis_name="s", num_cores=NC, num_subcores=NS)


@functools.partial(
    pl.kernel,
    out_type=jax.ShapeDtypeStruct((NC, NP, D), jnp.float32),
    mesh=_sc_mesh,
    scratch_types=[
        pltpu.VMEM((2, CHUNK), jnp.int32),
        pltpu.VMEM((CHUNK, D), jnp.float32),
        pltpu.VMEM((CHUNK, D), jnp.float32),
        pltpu.VMEM_SHARED((NP, D), jnp.float32),
        pltpu.SemaphoreType.DMA,
    ],
)
def _sc_deg(dst_hbm, ones_hbm, zeros_hbm, deg_out, dstv, onesv, wbv, deg_sh,
            isem):
    cid = lax.axis_index("c")
    tid = lax.axis_index("s")
    wid = cid * NS + tid

    pltpu.sync_copy(ones_hbm, onesv)
    pltpu.sync_copy(zeros_hbm, wbv)
    for r in range(RPT // CHUNK):
        pltpu.sync_copy(wbv, deg_sh.at[pl.ds(tid * RPT + r * CHUNK, CHUNK)])
    plsc.subcore_barrier()

    def load_idx(j, slot):
        pltpu.async_copy(dst_hbm.at[wid, j], dstv.at[slot], isem)

    def wait_idx(j, slot):
        pltpu.make_async_copy(dst_hbm.at[wid, j], dstv.at[slot], isem).wait()

    load_idx(0, 0)

    def pair(p, _):
        j0 = 2 * p
        load_idx(j0 + 1, 1)
        wait_idx(j0, 0)
        pltpu.sync_copy(onesv, deg_sh.at[dstv.at[0]], add=True)

        @pl.when(p < CPW // 2 - 1)
        def _():
            load_idx(j0 + 2, 0)

        wait_idx(j0 + 1, 1)
        pltpu.sync_copy(onesv, deg_sh.at[dstv.at[1]], add=True)
        return 0
    lax.fori_loop(0, CPW // 2, pair, 0)
    plsc.subcore_barrier()

    for r in range(RPT // CHUNK):
        base = tid * RPT + r * CHUNK
        pltpu.sync_copy(deg_sh.at[pl.ds(base, CHUNK)], wbv)
        pltpu.sync_copy(wbv, deg_out.at[cid, pl.ds(base, CHUNK)])


@functools.partial(
    pl.kernel,
    out_type=jax.ShapeDtypeStruct((NC, NP, D), jnp.float32),
    mesh=_sc_mesh,
    scratch_types=[
        pltpu.VMEM((4, SUB), jnp.int32),
        pltpu.VMEM((4, SUB), jnp.int32),
        pltpu.VMEM((4, SUB, D), jnp.float32),
        pltpu.VMEM_SHARED((NP, D), jnp.float32),
        pltpu.SemaphoreType.DMA,
        pltpu.SemaphoreType.DMA,
        pltpu.SemaphoreType.DMA,
        pltpu.SemaphoreType.DMA,
    ],
)
def _sc_agg(hs_hbm, src_hbm, dst_hbm, zeros_hbm, acc_out,
            srcv, dstv, bufs, acc_sh, sem0, sem1, sem2, sem3):
    sems = (sem0, sem1, sem2, sem3)
    cid = lax.axis_index("c")
    tid = lax.axis_index("s")
    wid = cid * NS + tid

    pltpu.sync_copy(zeros_hbm, bufs.at[0])
    for r in range(RPT // SUB):
        pltpu.sync_copy(bufs.at[0], acc_sh.at[pl.ds(tid * RPT + r * SUB, SUB)])
    plsc.subcore_barrier()

    def load_and_gather(j, b):
        pltpu.sync_copy(src_hbm.at[wid, j], srcv.at[b])
        pltpu.sync_copy(dst_hbm.at[wid, j], dstv.at[b])
        pltpu.async_copy(hs_hbm.at[srcv.at[b]], bufs.at[b], sems[b])

    def wait_and_scatter(b):
        pltpu.make_async_copy(hs_hbm.at[srcv.at[b]], bufs.at[b],
                              sems[b]).wait()
        pltpu.sync_copy(bufs.at[b], acc_sh.at[dstv.at[b]], add=True)

    for b in range(4):
        load_and_gather(b, b)

    def ring(q, _):
        j0 = 4 * q
        for b in range(4):
            wait_and_scatter(b)

            @pl.when(q < NSUB // 4 - 1)
            def _():
                load_and_gather(j0 + 4 + b, b)
        return 0
    lax.fori_loop(0, NSUB // 4, ring, 0)
    plsc.subcore_barrier()

    for r in range(RPT // SUB):
        base = tid * RPT + r * SUB
        pltpu.sync_copy(acc_sh.at[pl.ds(base, SUB)], bufs.at[0])
        pltpu.sync_copy(bufs.at[0], acc_out.at[cid, pl.ds(base, SUB)])


def _tc_pro_body(x_ref, w_ref, deg0_ref, deg1_ref, o_ref, dis_ref):
    rows = pl.program_id(0) * RB + lax.broadcasted_iota(jnp.int32, (RB, 1), 0)
    degsum = deg0_ref[:, 0:1] + deg1_ref[:, 0:1] + 1.0
    dis = jnp.where(rows < N, lax.rsqrt(degsum), 0.0)
    h = jnp.dot(x_ref[...], w_ref[...], preferred_element_type=jnp.float32,
                precision=lax.Precision.HIGHEST)
    o_ref[...] = h * dis
    dis_ref[...] = dis


def _tc_mid_body(acc0_ref, acc1_ref, hs_ref, dis_ref, b_ref, w_ref, o_ref):
    dis = dis_ref[...]
    h = (acc0_ref[...] + acc1_ref[...] + hs_ref[...]) * dis + b_ref[...]
    h = jnp.maximum(h, 0.0)
    o_ref[...] = jnp.dot(h, w_ref[...], preferred_element_type=jnp.float32,
                         precision=lax.Precision.HIGHEST) * dis


def _tc_fin_body(acc0_ref, acc1_ref, hs_ref, dis_ref, b_ref,
                 wc_ref, bc_ref, o_ref):
    dis = dis_ref[...]
    h = (acc0_ref[...] + acc1_ref[...] + hs_ref[...]) * dis + b_ref[...]
    h = jnp.maximum(h, 0.0)
    logits = jnp.dot(h, wc_ref[...], preferred_element_type=jnp.float32,
                     precision=lax.Precision.HIGHEST) + bc_ref[...]
    m = jnp.max(logits, axis=1, keepdims=True)
    lse = jnp.log(jnp.sum(jnp.exp(logits - m), axis=1, keepdims=True)) + m
    o_ref[...] = logits - lse


def _row_spec(width):
    return pl.BlockSpec((RB, width), lambda i: (i, 0))


def _full_spec(h, w):
    return pl.BlockSpec((h, w), lambda i: (0, 0))


_tc_pro = pl.pallas_call(
    _tc_pro_body,
    grid=(GRID,),
    in_specs=[_row_spec(D), _full_spec(D, D), _row_spec(D), _row_spec(D)],
    out_specs=[_row_spec(D), _row_spec(1)],
    out_shape=[jax.ShapeDtypeStruct((NP, D), jnp.float32),
               jax.ShapeDtypeStruct((NP, 1), jnp.float32)],
)

_tc_mid = pl.pallas_call(
    _tc_mid_body,
    grid=(GRID,),
    in_specs=[_row_spec(D), _row_spec(D), _row_spec(D), _row_spec(1),
              _full_spec(1, D), _full_spec(D, D)],
    out_specs=_row_spec(D),
    out_shape=jax.ShapeDtypeStruct((NP, D), jnp.float32),
)

_tc_fin = pl.pallas_call(
    _tc_fin_body,
    grid=(GRID,),
    in_specs=[_row_spec(D), _row_spec(D), _row_spec(D), _row_spec(1),
              _full_spec(1, D), _full_spec(D, ODIM),
              _full_spec(1, ODIM)],
    out_specs=_row_spec(ODIM),
    out_shape=jax.ShapeDtypeStruct((NP, ODIM), jnp.float32),
)


def kernel(x, edge_index, W1, b1, W2, b2, Wc, bc):
    src = edge_index[0].astype(jnp.int32)
    dst = edge_index[1].astype(jnp.int32)
    npad = EP - src.shape[0]
    pad_idx = (jnp.arange(npad, dtype=jnp.int32) % (NP - N)) + N
    src_p = jnp.concatenate([src, pad_idx]).reshape(NW, NSUB, SUB)
    dst_f = jnp.concatenate([dst, pad_idx])
    dst_p = dst_f.reshape(NW, NSUB, SUB)
    dst_c = dst_f.reshape(NW, CPW, CHUNK)
    x_p = jnp.pad(x, ((0, NP - N), (0, 0)))
    ones16 = jnp.ones((CHUNK, D), jnp.float32)
    zeros16 = jnp.zeros((CHUNK, D), jnp.float32)
    zeros32 = jnp.zeros((SUB, D), jnp.float32)

    deg = _sc_deg(dst_c, ones16, zeros16)
    hs1, dis = _tc_pro(x_p, W1, deg[0], deg[1])
    acc1 = _sc_agg(hs1, src_p, dst_p, zeros32)
    hs2 = _tc_mid(acc1[0], acc1[1], hs1, dis, b1.reshape(1, D), W2)
    acc2 = _sc_agg(hs2, src_p, dst_p, zeros32)
    out = _tc_fin(acc2[0], acc2[1], hs2, dis, b2.reshape(1, D),
                  Wc, bc.reshape(1, ODIM))
    return out[:N]


# speedup vs baseline: 1.6182x; 1.6182x over previous
import functools

import jax
import jax.numpy as jnp
from jax import lax
from jax.experimental import pallas as pl
from jax.experimental.pallas import tpu as pltpu
from jax.experimental.pallas import tpu_sc as plsc

N = 10000
D = 128
ODIM = 64
NP = 10240
NC, NS = 2, 16
NW = NC * NS
CHUNK = 128
CPW = 80
SUB = 64
NSUB = 160
EPW = CPW * CHUNK
EP = NW * EPW
RPT = NP // NS
RB = 2048
GRID = NP // RB

_sc_mesh = plsc.VectorSubcoreMesh(
    core_axis_name="c", subcore_axis_name="s", num_cores=NC, num_subcores=NS)


@functools.partial(
    pl.kernel,
    out_type=jax.ShapeDtypeStruct((NC, NP, D), jnp.float32),
    mesh=_sc_mesh,
    scratch_types=[
        pltpu.VMEM((2, CHUNK), jnp.int32),
        pltpu.VMEM((CHUNK, D), jnp.float32),
        pltpu.VMEM((CHUNK, D), jnp.float32),
        pltpu.VMEM_SHARED((NP, D), jnp.float32),
        pltpu.SemaphoreType.DMA,
    ],
)
def _sc_deg(dst_hbm, ones_hbm, zeros_hbm, deg_out, dstv, onesv, wbv, deg_sh,
            isem):
    cid = lax.axis_index("c")
    tid = lax.axis_index("s")
    wid = cid * NS + tid

    pltpu.sync_copy(ones_hbm, onesv)
    pltpu.sync_copy(zeros_hbm, wbv)
    for r in range(RPT // CHUNK):
        pltpu.sync_copy(wbv, deg_sh.at[pl.ds(tid * RPT + r * CHUNK, CHUNK)])
    plsc.subcore_barrier()

    def load_idx(j, slot):
        pltpu.async_copy(dst_hbm.at[wid, j], dstv.at[slot], isem)

    def wait_idx(j, slot):
        pltpu.make_async_copy(dst_hbm.at[wid, j], dstv.at[slot], isem).wait()

    load_idx(0, 0)

    def pair(p, _):
        j0 = 2 * p
        load_idx(j0 + 1, 1)
        wait_idx(j0, 0)
        pltpu.sync_copy(onesv, deg_sh.at[dstv.at[0]], add=True)

        @pl.when(p < CPW // 2 - 1)
        def _():
            load_idx(j0 + 2, 0)

        wait_idx(j0 + 1, 1)
        pltpu.sync_copy(onesv, deg_sh.at[dstv.at[1]], add=True)
        return 0
    lax.fori_loop(0, CPW // 2, pair, 0)
    plsc.subcore_barrier()

    for r in range(RPT // CHUNK):
        base = tid * RPT + r * CHUNK
        pltpu.sync_copy(deg_sh.at[pl.ds(base, CHUNK)], wbv)
        pltpu.sync_copy(wbv, deg_out.at[cid, pl.ds(base, CHUNK)])


@functools.partial(
    pl.kernel,
    out_type=jax.ShapeDtypeStruct((NC, NP, D), jnp.float32),
    mesh=_sc_mesh,
    scratch_types=[
        pltpu.VMEM((4, 2, CHUNK), jnp.int32),
        pltpu.VMEM((CHUNK, D), jnp.float32),
        pltpu.VMEM((CHUNK, D), jnp.float32),
        pltpu.VMEM_SHARED((NP, D), jnp.float32),
        pltpu.SemaphoreType.DMA,
        pltpu.SemaphoreType.DMA,
        pltpu.SemaphoreType.DMA,
    ],
)
def _sc_agg(hs_hbm, idx_hbm, zeros_hbm, acc_out,
            idxv, buf0, buf1, acc_sh, gsem0, gsem1, isem):
    cid = lax.axis_index("c")
    tid = lax.axis_index("s")
    wid = cid * NS + tid

    pltpu.sync_copy(zeros_hbm, buf0)
    for r in range(RPT // CHUNK):
        pltpu.sync_copy(buf0, acc_sh.at[pl.ds(tid * RPT + r * CHUNK, CHUNK)])
    plsc.subcore_barrier()

    def load_idx(j, slot):
        pltpu.async_copy(idx_hbm.at[wid, j], idxv.at[slot], isem)

    def wait_idx(j, slot):
        pltpu.make_async_copy(idx_hbm.at[wid, j], idxv.at[slot], isem).wait()

    def start_gather(slot, buf, sem):
        pltpu.async_copy(hs_hbm.at[idxv.at[slot, 0]], buf, sem)

    def wait_gather(slot, buf, sem):
        pltpu.make_async_copy(hs_hbm.at[idxv.at[slot, 0]], buf, sem).wait()

    def scatter(slot, buf):
        pltpu.sync_copy(buf, acc_sh.at[idxv.at[slot, 1]], add=True)

    for s in range(4):
        load_idx(s, s)
    wait_idx(0, 0)
    start_gather(0, buf0, gsem0)

    NQ = CPW // 4

    def quad(q, _):
        j0 = 4 * q
        not_last = q < NQ - 1

        wait_idx(j0 + 1, 1)
        start_gather(1, buf1, gsem1)
        wait_gather(0, buf0, gsem0)
        scatter(0, buf0)

        @pl.when(not_last)
        def _():
            load_idx(j0 + 4, 0)

        wait_idx(j0 + 2, 2)
        start_gather(2, buf0, gsem0)
        wait_gather(1, buf1, gsem1)
        scatter(1, buf1)

        @pl.when(not_last)
        def _():
            load_idx(j0 + 5, 1)

        wait_idx(j0 + 3, 3)
        start_gather(3, buf1, gsem1)
        wait_gather(2, buf0, gsem0)
        scatter(2, buf0)

        @pl.when(not_last)
        def _():
            load_idx(j0 + 6, 2)
            wait_idx(j0 + 4, 0)
            start_gather(0, buf0, gsem0)

        wait_gather(3, buf1, gsem1)
        scatter(3, buf1)

        @pl.when(not_last)
        def _():
            load_idx(j0 + 7, 3)
        return 0
    lax.fori_loop(0, NQ, quad, 0)
    plsc.subcore_barrier()

    for r in range(RPT // CHUNK):
        base = tid * RPT + r * CHUNK
        pltpu.sync_copy(acc_sh.at[pl.ds(base, CHUNK)], buf0)
        pltpu.sync_copy(buf0, acc_out.at[cid, pl.ds(base, CHUNK)])


def _tc_pro_body(x_ref, w_ref, deg0_ref, deg1_ref, o_ref, dis_ref):
    rows = pl.program_id(0) * RB + lax.broadcasted_iota(jnp.int32, (RB, 1), 0)
    degsum = deg0_ref[:, 0:1] + deg1_ref[:, 0:1] + 1.0
    dis = jnp.where(rows < N, lax.rsqrt(degsum), 0.0)
    h = jnp.dot(x_ref[...], w_ref[...], preferred_element_type=jnp.float32,
                precision=lax.Precision.HIGHEST)
    o_ref[...] = h * dis
    dis_ref[...] = dis


def _tc_mid_body(acc0_ref, acc1_ref, hs_ref, dis_ref, b_ref, w_ref, o_ref):
    dis = dis_ref[...]
    h = (acc0_ref[...] + acc1_ref[...] + hs_ref[...]) * dis + b_ref[...]
    h = jnp.maximum(h, 0.0)
    o_ref[...] = jnp.dot(h, w_ref[...], preferred_element_type=jnp.float32,
                         precision=lax.Precision.HIGHEST) * dis


def _tc_fin_body(acc0_ref, acc1_ref, hs_ref, dis_ref, b_ref,
                 wc_ref, bc_ref, o_ref):
    dis = dis_ref[...]
    h = (acc0_ref[...] + acc1_ref[...] + hs_ref[...]) * dis + b_ref[...]
    h = jnp.maximum(h, 0.0)
    logits = jnp.dot(h, wc_ref[...], preferred_element_type=jnp.float32,
                     precision=lax.Precision.HIGHEST) + bc_ref[...]
    m = jnp.max(logits, axis=1, keepdims=True)
    lse = jnp.log(jnp.sum(jnp.exp(logits - m), axis=1, keepdims=True)) + m
    o_ref[...] = logits - lse


def _row_spec(width):
    return pl.BlockSpec((RB, width), lambda i: (i, 0))


def _full_spec(h, w):
    return pl.BlockSpec((h, w), lambda i: (0, 0))


_tc_pro = pl.pallas_call(
    _tc_pro_body,
    grid=(GRID,),
    in_specs=[_row_spec(D), _full_spec(D, D), _row_spec(D), _row_spec(D)],
    out_specs=[_row_spec(D), _row_spec(1)],
    out_shape=[jax.ShapeDtypeStruct((NP, D), jnp.float32),
               jax.ShapeDtypeStruct((NP, 1), jnp.float32)],
)

_tc_mid = pl.pallas_call(
    _tc_mid_body,
    grid=(GRID,),
    in_specs=[_row_spec(D), _row_spec(D), _row_spec(D), _row_spec(1),
              _full_spec(1, D), _full_spec(D, D)],
    out_specs=_row_spec(D),
    out_shape=jax.ShapeDtypeStruct((NP, D), jnp.float32),
)

_tc_fin = pl.pallas_call(
    _tc_fin_body,
    grid=(GRID,),
    in_specs=[_row_spec(D), _row_spec(D), _row_spec(D), _row_spec(1),
              _full_spec(1, D), _full_spec(D, ODIM),
              _full_spec(1, ODIM)],
    out_specs=_row_spec(ODIM),
    out_shape=jax.ShapeDtypeStruct((NP, ODIM), jnp.float32),
)


def kernel(x, edge_index, W1, b1, W2, b2, Wc, bc):
    src = edge_index[0].astype(jnp.int32)
    dst = edge_index[1].astype(jnp.int32)
    npad = EP - src.shape[0]
    pad_idx = (jnp.arange(npad, dtype=jnp.int32) % (NP - N)) + N
    src_f = jnp.concatenate([src, pad_idx]).reshape(NW, CPW, 1, CHUNK)
    dst_f = jnp.concatenate([dst, pad_idx]).reshape(NW, CPW, 1, CHUNK)
    idx_p = jnp.concatenate([src_f, dst_f], axis=2)
    dst_c = dst_f.reshape(NW, CPW, CHUNK)
    x_p = jnp.pad(x, ((0, NP - N), (0, 0)))
    ones16 = jnp.ones((CHUNK, D), jnp.float32)
    zeros16 = jnp.zeros((CHUNK, D), jnp.float32)

    deg = _sc_deg(dst_c, ones16, zeros16)
    hs1, dis = _tc_pro(x_p, W1, deg[0], deg[1])
    acc1 = _sc_agg(hs1, idx_p, zeros16)
    hs2 = _tc_mid(acc1[0], acc1[1], hs1, dis, b1.reshape(1, D), W2)
    acc2 = _sc_agg(hs2, idx_p, zeros16)
    out = _tc_fin(acc2[0], acc2[1], hs2, dis, b2.reshape(1, D),
                  Wc, bc.reshape(1, ODIM))
    return out[:N]
